# baseline (speedup 1.0000x reference)
import jax
import jax.numpy as jnp
from jax import lax
from jax.experimental import pallas as pl
from jax.experimental.pallas import tpu as pltpu
from jax.experimental.pallas import tpu_sc as plsc

_SEQ_LEN = 16384
_VOCAB = 1000
_PAD = 0

_NC = 2
_NS = 16
_L = 16
_NW = _NC * _NS
_RPW = _SEQ_LEN // _NW
_WPW = _RPW * _VOCAB

_ZREG = 512_000
_ZSLICE = _ZREG // _NS
_NZDMA = _WPW // _ZREG

_NIDX = 128
_NXFER = _RPW // _NIDX

_TOTAL = _SEQ_LEN * _VOCAB


def _one_hot_body(seq_hbm, vals_hbm, out_hbm, seq_v, vals_v, zbuf, idx_v,
                  src_v, zeros_sh, zsem, ssem):
    cid = lax.axis_index("c")
    sid = lax.axis_index("s")
    wid = sid * _NC + cid
    base = wid * _RPW

    zeros16 = jnp.zeros((_L,), jnp.float32)

    def zbody(i, carry):
        for u in range(8):
            zbuf[pl.ds((i * 8 + u) * _L, _L)] = zeros16
        return carry

    lax.fori_loop(0, _ZSLICE // (_L * 8), zbody, 0)
    pltpu.sync_copy(zbuf, zeros_sh.at[pl.ds(sid * _ZSLICE, _ZSLICE)])
    plsc.subcore_barrier()

    zcopies = [
        pltpu.async_copy(
            zeros_sh,
            out_hbm.at[pl.ds(base * _VOCAB + k * _ZREG, _ZREG)],
            zsem,
        )
        for k in range(_NZDMA)
    ]

    pltpu.sync_copy(seq_hbm.at[pl.ds(base, _RPW)], seq_v)
    pltpu.sync_copy(vals_hbm.at[pl.ds(base, _RPW)], vals_v)
    lane = lax.iota(jnp.int32, _L)
    for t in range(_RPW // _L):
        seq16 = seq_v[pl.ds(t * _L, _L)]
        v16 = vals_v[pl.ds(t * _L, _L)]
        gidx = (base + t * _L + lane) * _VOCAB + seq16
        val = jnp.where(seq16 != _PAD, v16, jnp.zeros((_L,), jnp.float32))
        j, c0 = divmod(t * _L, _NIDX)
        idx_v[j, pl.ds(c0, _L)] = gidx
        src_v[j, pl.ds(c0, _L)] = val

    for cp in zcopies:
        cp.wait()

    scopies = [
        pltpu.async_copy(src_v.at[j], out_hbm.at[idx_v.at[j]], ssem)
        for j in range(_NXFER)
    ]
    for cp in scopies:
        cp.wait()


@jax.jit
def kernel(sequence, vals):
    mesh = plsc.VectorSubcoreMesh(core_axis_name="c", subcore_axis_name="s")
    flat = pl.kernel(
        _one_hot_body,
        mesh=mesh,
        compiler_params=pltpu.CompilerParams(needs_layout_passes=False),
        out_type=jax.ShapeDtypeStruct((_TOTAL,), jnp.float32),
        scratch_types=[
            pltpu.VMEM((_RPW,), jnp.int32),
            pltpu.VMEM((_RPW,), jnp.float32),
            pltpu.VMEM((_ZSLICE,), jnp.float32),
            pltpu.VMEM((_NXFER, _NIDX), jnp.int32),
            pltpu.VMEM((_NXFER, _NIDX), jnp.float32),
            pltpu.VMEM_SHARED((_ZREG,), jnp.float32),
            pltpu.SemaphoreType.DMA,
            pltpu.SemaphoreType.DMA,
        ],
    )(sequence, vals)
    return flat.reshape(_SEQ_LEN, _VOCAB)
